# baseline (device time: 85532 ns/iter reference)
import jax
import jax.numpy as jnp
from jax import lax
from jax.experimental import pallas as pl
from jax.experimental.pallas import tpu as pltpu

N_Z = 4


def kernel(partial, resid, gamma):
    _, m, d = partial.shape

    def body(partial_ref, resid_ref, gamma_ref, out_ref,
             comm_ref, send_sems, recv_sems):
        my_x = lax.axis_index("x")
        my_y = lax.axis_index("y")
        my_z = lax.axis_index("z")
        fwd = (my_z + 1) % N_Z
        bwd = (my_z - 1) % N_Z

        barrier_sem = pltpu.get_barrier_semaphore()
        for nbr in [fwd, bwd]:
            pl.semaphore_signal(
                barrier_sem, inc=1,
                device_id=(my_x, my_y, nbr),
                device_id_type=pl.DeviceIdType.MESH,
            )
        pl.semaphore_wait(barrier_sem, 2)

        comm_ref[0, :, :] = partial_ref[0, :, :].astype(jnp.bfloat16)

        for h in range(N_Z - 1):
            rdma = pltpu.make_async_remote_copy(
                src_ref=comm_ref.at[h],
                dst_ref=comm_ref.at[h + 1],
                send_sem=send_sems.at[h],
                recv_sem=recv_sems.at[h],
                device_id=(my_x, my_y, fwd),
                device_id_type=pl.DeviceIdType.MESH,
            )
            rdma.start()
            rdma.wait()

        acc = (comm_ref[0, :, :].astype(jnp.float32)
               + comm_ref[1, :, :].astype(jnp.float32)
               + comm_ref[2, :, :].astype(jnp.float32)
               + comm_ref[3, :, :].astype(jnp.float32))
        y = acc + resid_ref[:, :]
        ms = jnp.mean(y * y, axis=-1, keepdims=True)
        inv_rms = lax.rsqrt(ms + 1e-6)
        out_ref[:, :] = y * inv_rms * gamma_ref[0, :]

    return pl.pallas_call(
        body,
        out_shape=jax.ShapeDtypeStruct((m, d), jnp.float32),
        in_specs=[
            pl.BlockSpec(memory_space=pltpu.VMEM),
            pl.BlockSpec(memory_space=pltpu.VMEM),
            pl.BlockSpec(memory_space=pltpu.VMEM),
        ],
        out_specs=pl.BlockSpec(memory_space=pltpu.VMEM),
        scratch_shapes=[
            pltpu.VMEM((N_Z, m, d), jnp.bfloat16),
            pltpu.SemaphoreType.DMA((N_Z - 1,)),
            pltpu.SemaphoreType.DMA((N_Z - 1,)),
        ],
        compiler_params=pltpu.CompilerParams(collective_id=0),
    )(partial, resid, gamma.reshape(1, d))


# device time: 47286 ns/iter; 1.8088x vs baseline; 1.8088x over previous
import jax
import jax.numpy as jnp
from jax import lax
from jax.experimental import pallas as pl
from jax.experimental.pallas import tpu as pltpu

N_Z = 4
Q = 256

S1, S2, S3, SCX, SCY, SCD = range(6)


def kernel(partial, resid, gamma):
    _, m, d = partial.shape

    def body(partial_ref, resid_ref, gamma_ref, out_ref,
             my_buf, s1_buf, x2out, x2in, tot_buf, ln_buf,
             qx_buf, qy_buf, qd_buf, send_sems, recv_sems):
        my_x = lax.axis_index("x")
        my_y = lax.axis_index("y")
        my_z = lax.axis_index("z")
        col_id = 2 * my_x + my_y
        q0 = col_id * Q

        is_inner = jnp.logical_or(my_z == 1, my_z == 2)
        zpair = my_z ^ 1
        zmid = 3 - my_z
        xnbr = (1 - my_x, my_y, my_z)
        ynbr = (my_x, 1 - my_y, my_z)
        diag = (1 - my_x, 1 - my_y, my_z)

        barrier_sem = pltpu.get_barrier_semaphore()
        for dev in [(my_x, my_y, zpair), (my_x, my_y, zmid), xnbr, ynbr, diag]:
            pl.semaphore_signal(
                barrier_sem, inc=1,
                device_id=dev, device_id_type=pl.DeviceIdType.MESH,
            )
        pl.semaphore_wait(barrier_sem, 5)

        my_buf[:, :] = partial_ref[0, pl.ds(q0, Q), :].astype(jnp.bfloat16)

        s1 = pltpu.make_async_remote_copy(
            src_ref=my_buf, dst_ref=s1_buf,
            send_sem=send_sems.at[S1], recv_sem=recv_sems.at[S1],
            device_id=(my_x, my_y, zpair), device_id_type=pl.DeviceIdType.MESH,
        )

        @pl.when(jnp.logical_not(is_inner))
        def _():
            s1.start()
            s1.wait_send()

        @pl.when(is_inner)
        def _():
            s1.wait_recv()
            x2out[:, :] = my_buf[:, :] + s1_buf[:, :]
            s2 = pltpu.make_async_remote_copy(
                src_ref=x2out, dst_ref=x2in,
                send_sem=send_sems.at[S2], recv_sem=recv_sems.at[S2],
                device_id=(my_x, my_y, zmid),
                device_id_type=pl.DeviceIdType.MESH,
            )
            s2.start()
            s2.wait()
            tot_buf[:, :] = x2out[:, :] + x2in[:, :]

        s3 = pltpu.make_async_remote_copy(
            src_ref=tot_buf, dst_ref=tot_buf,
            send_sem=send_sems.at[S3], recv_sem=recv_sems.at[S3],
            device_id=(my_x, my_y, zpair), device_id_type=pl.DeviceIdType.MESH,
        )

        @pl.when(is_inner)
        def _():
            s3.start()
            s3.wait_send()

        @pl.when(jnp.logical_not(is_inner))
        def _():
            s3.wait_recv()

        y = tot_buf[:, :].astype(jnp.float32) + resid_ref[pl.ds(q0, Q), :]
        ms = jnp.mean(y * y, axis=-1, keepdims=True)
        outq = y * lax.rsqrt(ms + 1e-6) * gamma_ref[0, :]
        out_ref[pl.ds(q0, Q), :] = outq
        ln_buf[:, :] = outq.astype(jnp.bfloat16)

        rx = pltpu.make_async_remote_copy(
            src_ref=ln_buf, dst_ref=qx_buf,
            send_sem=send_sems.at[SCX], recv_sem=recv_sems.at[SCX],
            device_id=xnbr, device_id_type=pl.DeviceIdType.MESH,
        )
        ry = pltpu.make_async_remote_copy(
            src_ref=ln_buf, dst_ref=qy_buf,
            send_sem=send_sems.at[SCY], recv_sem=recv_sems.at[SCY],
            device_id=ynbr, device_id_type=pl.DeviceIdType.MESH,
        )
        rd = pltpu.make_async_remote_copy(
            src_ref=ln_buf, dst_ref=qd_buf,
            send_sem=send_sems.at[SCD], recv_sem=recv_sems.at[SCD],
            device_id=diag, device_id_type=pl.DeviceIdType.MESH,
        )
        rx.start()
        ry.start()
        rd.start()

        qx_idx = 2 * (1 - my_x) + my_y
        qy_idx = 2 * my_x + (1 - my_y)
        qd_idx = 2 * (1 - my_x) + (1 - my_y)
        rx.wait_recv()
        out_ref[pl.ds(qx_idx * Q, Q), :] = qx_buf[:, :].astype(jnp.float32)
        ry.wait_recv()
        out_ref[pl.ds(qy_idx * Q, Q), :] = qy_buf[:, :].astype(jnp.float32)
        rd.wait_recv()
        out_ref[pl.ds(qd_idx * Q, Q), :] = qd_buf[:, :].astype(jnp.float32)
        rx.wait_send()
        ry.wait_send()
        rd.wait_send()

    qbuf = pltpu.VMEM((Q, d), jnp.bfloat16)
    return pl.pallas_call(
        body,
        out_shape=jax.ShapeDtypeStruct((m, d), jnp.float32),
        in_specs=[
            pl.BlockSpec(memory_space=pltpu.VMEM),
            pl.BlockSpec(memory_space=pltpu.VMEM),
            pl.BlockSpec(memory_space=pltpu.VMEM),
        ],
        out_specs=pl.BlockSpec(memory_space=pltpu.VMEM),
        scratch_shapes=[
            qbuf,
            qbuf,
            qbuf,
            qbuf,
            qbuf,
            qbuf,
            qbuf,
            qbuf,
            qbuf,
            pltpu.SemaphoreType.DMA((6,)),
            pltpu.SemaphoreType.DMA((6,)),
        ],
        compiler_params=pltpu.CompilerParams(collective_id=0),
    )(partial, resid, gamma.reshape(1, d))


# device time: 38690 ns/iter; 2.2107x vs baseline; 1.2222x over previous
import jax
import jax.numpy as jnp
from jax import lax
from jax.experimental import pallas as pl
from jax.experimental.pallas import tpu as pltpu

Q = 256
K_SUB = 2
R = Q // K_SUB

S1, S2, S3, SCX, SCY, SCD = range(6)
NSEM = 6


def kernel(partial, resid, gamma):
    _, m, d = partial.shape

    def body(partial_ref, resid_ref, gamma_ref, out_ref,
             my_buf, s1_buf, x2out, x2in, ln_buf,
             qx_buf, qy_buf, qd_buf, send_sems, recv_sems):
        my_x = lax.axis_index("x")
        my_y = lax.axis_index("y")
        my_z = lax.axis_index("z")
        col_id = 2 * my_x + my_y
        q0 = col_id * Q

        is_inner = jnp.logical_or(my_z == 1, my_z == 2)
        zpair = my_z ^ 1
        zmid = 3 - my_z
        xnbr = (1 - my_x, my_y, my_z)
        ynbr = (my_x, 1 - my_y, my_z)
        diag = (1 - my_x, 1 - my_y, my_z)

        barrier_sem = pltpu.get_barrier_semaphore()
        for dev in [(my_x, my_y, zpair), (my_x, my_y, zmid), xnbr, ynbr, diag]:
            pl.semaphore_signal(
                barrier_sem, inc=1,
                device_id=dev, device_id_type=pl.DeviceIdType.MESH,
            )
        pl.semaphore_wait(barrier_sem, 5)

        my_buf[:, :, :] = jnp.reshape(
            partial_ref[0, pl.ds(q0, Q), :].astype(jnp.bfloat16), (K_SUB, R, d)
        )

        def mk(src, dst, slot, dev):
            return pltpu.make_async_remote_copy(
                src_ref=src, dst_ref=dst,
                send_sem=send_sems.at[slot], recv_sem=recv_sems.at[slot],
                device_id=dev, device_id_type=pl.DeviceIdType.MESH,
            )

        s1 = [mk(my_buf.at[s], s1_buf.at[s], s * NSEM + S1,
                 (my_x, my_y, zpair)) for s in range(K_SUB)]
        s2 = [mk(x2out.at[s], x2in.at[s], s * NSEM + S2,
                 (my_x, my_y, zmid)) for s in range(K_SUB)]
        s3 = [mk(ln_buf.at[s], ln_buf.at[s], s * NSEM + S3,
                 (my_x, my_y, zpair)) for s in range(K_SUB)]
        rx = [mk(ln_buf.at[s], qx_buf.at[s], s * NSEM + SCX, xnbr)
              for s in range(K_SUB)]
        ry = [mk(ln_buf.at[s], qy_buf.at[s], s * NSEM + SCY, ynbr)
              for s in range(K_SUB)]
        rd = [mk(ln_buf.at[s], qd_buf.at[s], s * NSEM + SCD, diag)
              for s in range(K_SUB)]

        @pl.when(jnp.logical_not(is_inner))
        def _():
            for s in range(K_SUB):
                s1[s].start()
            for s in range(K_SUB):
                s3[s].wait_recv()
                out_ref[pl.ds(q0 + s * R, R), :] = \
                    ln_buf[s, :, :].astype(jnp.float32)
                rx[s].start()
                ry[s].start()
                rd[s].start()
            for s in range(K_SUB):
                s1[s].wait_send()

        @pl.when(is_inner)
        def _():
            for s in range(K_SUB):
                s1[s].wait_recv()
                x2out[s, :, :] = my_buf[s, :, :] + s1_buf[s, :, :]
                s2[s].start()
            for s in range(K_SUB):
                s2[s].wait()
                tot = x2out[s, :, :] + x2in[s, :, :]
                y = tot.astype(jnp.float32) + resid_ref[pl.ds(q0 + s * R, R), :]
                ms = jnp.mean(y * y, axis=-1, keepdims=True)
                outq = y * lax.rsqrt(ms + 1e-6) * gamma_ref[0, :]
                out_ref[pl.ds(q0 + s * R, R), :] = outq
                ln_buf[s, :, :] = outq.astype(jnp.bfloat16)
                s3[s].start()
                rx[s].start()
                ry[s].start()
                rd[s].start()
            for s in range(K_SUB):
                s3[s].wait_send()

        qx_r0 = (2 * (1 - my_x) + my_y) * Q
        qy_r0 = (2 * my_x + (1 - my_y)) * Q
        qd_r0 = (2 * (1 - my_x) + (1 - my_y)) * Q
        for s in range(K_SUB):
            rx[s].wait_recv()
            out_ref[pl.ds(qx_r0 + s * R, R), :] = \
                qx_buf[s, :, :].astype(jnp.float32)
            ry[s].wait_recv()
            out_ref[pl.ds(qy_r0 + s * R, R), :] = \
                qy_buf[s, :, :].astype(jnp.float32)
            rd[s].wait_recv()
            out_ref[pl.ds(qd_r0 + s * R, R), :] = \
                qd_buf[s, :, :].astype(jnp.float32)
        for s in range(K_SUB):
            rx[s].wait_send()
            ry[s].wait_send()
            rd[s].wait_send()

    sub_buf = pltpu.VMEM((K_SUB, R, d), jnp.bfloat16)
    return pl.pallas_call(
        body,
        out_shape=jax.ShapeDtypeStruct((m, d), jnp.float32),
        in_specs=[
            pl.BlockSpec(memory_space=pltpu.VMEM),
            pl.BlockSpec(memory_space=pltpu.VMEM),
            pl.BlockSpec(memory_space=pltpu.VMEM),
        ],
        out_specs=pl.BlockSpec(memory_space=pltpu.VMEM),
        scratch_shapes=[
            sub_buf,
            sub_buf,
            sub_buf,
            sub_buf,
            sub_buf,
            sub_buf,
            sub_buf,
            sub_buf,
            pltpu.SemaphoreType.DMA((K_SUB * NSEM,)),
            pltpu.SemaphoreType.DMA((K_SUB * NSEM,)),
        ],
        compiler_params=pltpu.CompilerParams(collective_id=0),
    )(partial, resid, gamma.reshape(1, d))


# device time: 36868 ns/iter; 2.3200x vs baseline; 1.0494x over previous
import jax
import jax.numpy as jnp
from jax import lax
from jax.experimental import pallas as pl
from jax.experimental.pallas import tpu as pltpu

Q = 256
K_SUB = 4
R = Q // K_SUB

S1, S2, S3, SCX, SCY, SCD = range(6)
NSEM = 6


def kernel(partial, resid, gamma):
    _, m, d = partial.shape

    def body(partial_ref, resid_ref, gamma_ref, out_ref,
             my_buf, s1_buf, x2out, x2in, ln_buf,
             qx_buf, qy_buf, qd_buf, send_sems, recv_sems):
        my_x = lax.axis_index("x")
        my_y = lax.axis_index("y")
        my_z = lax.axis_index("z")
        col_id = 2 * my_x + my_y
        q0 = col_id * Q

        is_inner = jnp.logical_or(my_z == 1, my_z == 2)
        zpair = my_z ^ 1
        zmid = 3 - my_z
        xnbr = (1 - my_x, my_y, my_z)
        ynbr = (my_x, 1 - my_y, my_z)
        diag = (1 - my_x, 1 - my_y, my_z)

        barrier_sem = pltpu.get_barrier_semaphore()
        for dev in [(my_x, my_y, zpair), (my_x, my_y, zmid), xnbr, ynbr, diag]:
            pl.semaphore_signal(
                barrier_sem, inc=1,
                device_id=dev, device_id_type=pl.DeviceIdType.MESH,
            )
        pl.semaphore_wait(barrier_sem, 5)

        my_buf[:, :, :] = jnp.reshape(
            partial_ref[0, pl.ds(q0, Q), :].astype(jnp.bfloat16), (K_SUB, R, d)
        )

        def mk(src, dst, slot, dev):
            return pltpu.make_async_remote_copy(
                src_ref=src, dst_ref=dst,
                send_sem=send_sems.at[slot], recv_sem=recv_sems.at[slot],
                device_id=dev, device_id_type=pl.DeviceIdType.MESH,
            )

        s1 = [mk(my_buf.at[s], s1_buf.at[s], s * NSEM + S1,
                 (my_x, my_y, zpair)) for s in range(K_SUB)]
        s2 = [mk(x2out.at[s], x2in.at[s], s * NSEM + S2,
                 (my_x, my_y, zmid)) for s in range(K_SUB)]
        s3 = [mk(ln_buf.at[s], ln_buf.at[s], s * NSEM + S3,
                 (my_x, my_y, zpair)) for s in range(K_SUB)]
        rx = [mk(ln_buf.at[s], qx_buf.at[s], s * NSEM + SCX, xnbr)
              for s in range(K_SUB)]
        ry = [mk(ln_buf.at[s], qy_buf.at[s], s * NSEM + SCY, ynbr)
              for s in range(K_SUB)]
        rd = [mk(ln_buf.at[s], qd_buf.at[s], s * NSEM + SCD, diag)
              for s in range(K_SUB)]

        @pl.when(jnp.logical_not(is_inner))
        def _():
            for s in range(K_SUB):
                s1[s].start()
            for s in range(K_SUB):
                s3[s].wait_recv()
                out_ref[pl.ds(q0 + s * R, R), :] = \
                    ln_buf[s, :, :].astype(jnp.float32)
                rx[s].start()
                ry[s].start()
                rd[s].start()
            for s in range(K_SUB):
                s1[s].wait_send()

        @pl.when(is_inner)
        def _():
            for s in range(K_SUB):
                s1[s].wait_recv()
                x2out[s, :, :] = my_buf[s, :, :] + s1_buf[s, :, :]
                s2[s].start()
            for s in range(K_SUB):
                s2[s].wait()
                tot = x2out[s, :, :] + x2in[s, :, :]
                y = tot.astype(jnp.float32) + resid_ref[pl.ds(q0 + s * R, R), :]
                ms = jnp.mean(y * y, axis=-1, keepdims=True)
                outq = y * lax.rsqrt(ms + 1e-6) * gamma_ref[0, :]
                out_ref[pl.ds(q0 + s * R, R), :] = outq
                ln_buf[s, :, :] = outq.astype(jnp.bfloat16)
                s3[s].start()
                rx[s].start()
                ry[s].start()
                rd[s].start()
            for s in range(K_SUB):
                s3[s].wait_send()

        qx_r0 = (2 * (1 - my_x) + my_y) * Q
        qy_r0 = (2 * my_x + (1 - my_y)) * Q
        qd_r0 = (2 * (1 - my_x) + (1 - my_y)) * Q
        for s in range(K_SUB):
            rx[s].wait_recv()
            out_ref[pl.ds(qx_r0 + s * R, R), :] = \
                qx_buf[s, :, :].astype(jnp.float32)
            ry[s].wait_recv()
            out_ref[pl.ds(qy_r0 + s * R, R), :] = \
                qy_buf[s, :, :].astype(jnp.float32)
            rd[s].wait_recv()
            out_ref[pl.ds(qd_r0 + s * R, R), :] = \
                qd_buf[s, :, :].astype(jnp.float32)
        for s in range(K_SUB):
            rx[s].wait_send()
            ry[s].wait_send()
            rd[s].wait_send()

    sub_buf = pltpu.VMEM((K_SUB, R, d), jnp.bfloat16)
    return pl.pallas_call(
        body,
        out_shape=jax.ShapeDtypeStruct((m, d), jnp.float32),
        in_specs=[
            pl.BlockSpec(memory_space=pltpu.VMEM),
            pl.BlockSpec(memory_space=pltpu.VMEM),
            pl.BlockSpec(memory_space=pltpu.VMEM),
        ],
        out_specs=pl.BlockSpec(memory_space=pltpu.VMEM),
        scratch_shapes=[
            sub_buf,
            sub_buf,
            sub_buf,
            sub_buf,
            sub_buf,
            sub_buf,
            sub_buf,
            sub_buf,
            pltpu.SemaphoreType.DMA((K_SUB * NSEM,)),
            pltpu.SemaphoreType.DMA((K_SUB * NSEM,)),
        ],
        compiler_params=pltpu.CompilerParams(collective_id=0),
    )(partial, resid, gamma.reshape(1, d))


# device time: 34088 ns/iter; 2.5092x vs baseline; 1.0816x over previous
import jax
import jax.numpy as jnp
from jax import lax
from jax.experimental import pallas as pl
from jax.experimental.pallas import tpu as pltpu

Q = 256
K_SUB = 4
R = Q // K_SUB

S1, SMID, SNEAR, SFAR, SCX, SCY, SCD = range(7)
NSEM = 7


def kernel(partial, resid, gamma):
    _, m, d = partial.shape

    def body(partial_ref, resid_ref, gamma_ref, out_ref,
             my_buf, s1_buf, p_out, p_in, pn_buf, pf_buf, ln_buf,
             qx_buf, qy_buf, qd_buf, send_sems, recv_sems):
        my_x = lax.axis_index("x")
        my_y = lax.axis_index("y")
        my_z = lax.axis_index("z")
        col_id = 2 * my_x + my_y
        q0 = col_id * Q

        is_inner = jnp.logical_or(my_z == 1, my_z == 2)
        zpair = my_z ^ 1
        zmid = 3 - my_z
        zfar = my_z ^ 2
        xnbr = (1 - my_x, my_y, my_z)
        ynbr = (my_x, 1 - my_y, my_z)
        diag = (1 - my_x, 1 - my_y, my_z)

        barrier_sem = pltpu.get_barrier_semaphore()
        for dev in [(my_x, my_y, zpair), (my_x, my_y, zmid),
                    (my_x, my_y, zfar), xnbr, ynbr, diag]:
            pl.semaphore_signal(
                barrier_sem, inc=1,
                device_id=dev, device_id_type=pl.DeviceIdType.MESH,
            )
        pl.semaphore_wait(barrier_sem, 6)

        my_buf[:, :, :] = jnp.reshape(
            partial_ref[0, pl.ds(q0, Q), :].astype(jnp.bfloat16), (K_SUB, R, d)
        )

        def mk(src, dst, slot, dev):
            return pltpu.make_async_remote_copy(
                src_ref=src, dst_ref=dst,
                send_sem=send_sems.at[slot], recv_sem=recv_sems.at[slot],
                device_id=dev, device_id_type=pl.DeviceIdType.MESH,
            )

        s1 = [mk(my_buf.at[s], s1_buf.at[s], s * NSEM + S1,
                 (my_x, my_y, zpair)) for s in range(K_SUB)]
        mid = [mk(p_out.at[s], p_in.at[s], s * NSEM + SMID,
                  (my_x, my_y, zmid)) for s in range(K_SUB)]
        near = [mk(p_out.at[s], pn_buf.at[s], s * NSEM + SNEAR,
                   (my_x, my_y, zpair)) for s in range(K_SUB)]
        far = [mk(p_out.at[s], pf_buf.at[s], s * NSEM + SFAR,
                  (my_x, my_y, zfar)) for s in range(K_SUB)]
        rx = [mk(ln_buf.at[s], qx_buf.at[s], s * NSEM + SCX, xnbr)
              for s in range(K_SUB)]
        ry = [mk(ln_buf.at[s], qy_buf.at[s], s * NSEM + SCY, ynbr)
              for s in range(K_SUB)]
        rd = [mk(ln_buf.at[s], qd_buf.at[s], s * NSEM + SCD, diag)
              for s in range(K_SUB)]

        def ln_and_scatter(s, tot_bf16):
            y = tot_bf16.astype(jnp.float32) + resid_ref[pl.ds(q0 + s * R, R), :]
            ms = jnp.mean(y * y, axis=-1, keepdims=True)
            outq = y * lax.rsqrt(ms + 1e-6) * gamma_ref[0, :]
            out_ref[pl.ds(q0 + s * R, R), :] = outq
            ln_buf[s, :, :] = outq.astype(jnp.bfloat16)
            rx[s].start()
            ry[s].start()
            rd[s].start()

        @pl.when(jnp.logical_not(is_inner))
        def _():
            for s in range(K_SUB):
                s1[s].start()
            for s in range(K_SUB):
                near[s].wait_recv()
                far[s].wait_recv()
                ln_and_scatter(s, pn_buf[s, :, :] + pf_buf[s, :, :])
            for s in range(K_SUB):
                s1[s].wait_send()

        @pl.when(is_inner)
        def _():
            for s in range(K_SUB):
                s1[s].wait_recv()
                p_out[s, :, :] = my_buf[s, :, :] + s1_buf[s, :, :]
                mid[s].start()
                near[s].start()
                far[s].start()
            for s in range(K_SUB):
                mid[s].wait_recv()
                ln_and_scatter(s, p_out[s, :, :] + p_in[s, :, :])
            for s in range(K_SUB):
                mid[s].wait_send()
                near[s].wait_send()
                far[s].wait_send()

        qx_r0 = (2 * (1 - my_x) + my_y) * Q
        qy_r0 = (2 * my_x + (1 - my_y)) * Q
        qd_r0 = (2 * (1 - my_x) + (1 - my_y)) * Q
        for s in range(K_SUB):
            rx[s].wait_recv()
            out_ref[pl.ds(qx_r0 + s * R, R), :] = \
                qx_buf[s, :, :].astype(jnp.float32)
            ry[s].wait_recv()
            out_ref[pl.ds(qy_r0 + s * R, R), :] = \
                qy_buf[s, :, :].astype(jnp.float32)
            rd[s].wait_recv()
            out_ref[pl.ds(qd_r0 + s * R, R), :] = \
                qd_buf[s, :, :].astype(jnp.float32)
        for s in range(K_SUB):
            rx[s].wait_send()
            ry[s].wait_send()
            rd[s].wait_send()

    sub_buf = pltpu.VMEM((K_SUB, R, d), jnp.bfloat16)
    return pl.pallas_call(
        body,
        out_shape=jax.ShapeDtypeStruct((m, d), jnp.float32),
        in_specs=[
            pl.BlockSpec(memory_space=pltpu.VMEM),
            pl.BlockSpec(memory_space=pltpu.VMEM),
            pl.BlockSpec(memory_space=pltpu.VMEM),
        ],
        out_specs=pl.BlockSpec(memory_space=pltpu.VMEM),
        scratch_shapes=[
            sub_buf,
            sub_buf,
            sub_buf,
            sub_buf,
            sub_buf,
            sub_buf,
            sub_buf,
            sub_buf,
            sub_buf,
            sub_buf,
            pltpu.SemaphoreType.DMA((K_SUB * NSEM,)),
            pltpu.SemaphoreType.DMA((K_SUB * NSEM,)),
        ],
        compiler_params=pltpu.CompilerParams(collective_id=0),
    )(partial, resid, gamma.reshape(1, d))


# device time: 32383 ns/iter; 2.6413x vs baseline; 1.0527x over previous
import jax
import jax.numpy as jnp
from jax import lax
from jax.experimental import pallas as pl
from jax.experimental.pallas import tpu as pltpu

Q = 256
K_SUB = 8
R = Q // K_SUB

S1, SMID, SNEAR, SFAR, SCX, SCY, SCD = range(7)
NSEM = 7


def kernel(partial, resid, gamma):
    _, m, d = partial.shape

    def body(partial_ref, resid_ref, gamma_ref, out_ref,
             my_buf, s1_buf, p_out, p_in, pn_buf, pf_buf, ln_buf,
             qx_buf, qy_buf, qd_buf, send_sems, recv_sems):
        my_x = lax.axis_index("x")
        my_y = lax.axis_index("y")
        my_z = lax.axis_index("z")
        col_id = 2 * my_x + my_y
        q0 = col_id * Q

        is_inner = jnp.logical_or(my_z == 1, my_z == 2)
        zpair = my_z ^ 1
        zmid = 3 - my_z
        zfar = my_z ^ 2
        xnbr = (1 - my_x, my_y, my_z)
        ynbr = (my_x, 1 - my_y, my_z)
        diag = (1 - my_x, 1 - my_y, my_z)

        barrier_sem = pltpu.get_barrier_semaphore()
        for dev in [(my_x, my_y, zpair), (my_x, my_y, zmid),
                    (my_x, my_y, zfar), xnbr, ynbr, diag]:
            pl.semaphore_signal(
                barrier_sem, inc=1,
                device_id=dev, device_id_type=pl.DeviceIdType.MESH,
            )
        pl.semaphore_wait(barrier_sem, 6)

        my_buf[:, :, :] = jnp.reshape(
            partial_ref[0, pl.ds(q0, Q), :].astype(jnp.bfloat16), (K_SUB, R, d)
        )

        def mk(src, dst, slot, dev):
            return pltpu.make_async_remote_copy(
                src_ref=src, dst_ref=dst,
                send_sem=send_sems.at[slot], recv_sem=recv_sems.at[slot],
                device_id=dev, device_id_type=pl.DeviceIdType.MESH,
            )

        s1 = [mk(my_buf.at[s], s1_buf.at[s], s * NSEM + S1,
                 (my_x, my_y, zpair)) for s in range(K_SUB)]
        mid = [mk(p_out.at[s], p_in.at[s], s * NSEM + SMID,
                  (my_x, my_y, zmid)) for s in range(K_SUB)]
        near = [mk(p_out.at[s], pn_buf.at[s], s * NSEM + SNEAR,
                   (my_x, my_y, zpair)) for s in range(K_SUB)]
        far = [mk(p_out.at[s], pf_buf.at[s], s * NSEM + SFAR,
                  (my_x, my_y, zfar)) for s in range(K_SUB)]
        rx = [mk(ln_buf.at[s], qx_buf.at[s], s * NSEM + SCX, xnbr)
              for s in range(K_SUB)]
        ry = [mk(ln_buf.at[s], qy_buf.at[s], s * NSEM + SCY, ynbr)
              for s in range(K_SUB)]
        rd = [mk(ln_buf.at[s], qd_buf.at[s], s * NSEM + SCD, diag)
              for s in range(K_SUB)]

        def ln_and_scatter(s, tot_bf16):
            y = tot_bf16.astype(jnp.float32) + resid_ref[pl.ds(q0 + s * R, R), :]
            ms = jnp.mean(y * y, axis=-1, keepdims=True)
            outq = y * lax.rsqrt(ms + 1e-6) * gamma_ref[0, :]
            out_ref[pl.ds(q0 + s * R, R), :] = outq
            ln_buf[s, :, :] = outq.astype(jnp.bfloat16)
            rx[s].start()
            ry[s].start()
            rd[s].start()

        @pl.when(jnp.logical_not(is_inner))
        def _():
            for s in range(K_SUB):
                s1[s].start()
            for s in range(K_SUB):
                near[s].wait_recv()
                far[s].wait_recv()
                ln_and_scatter(s, pn_buf[s, :, :] + pf_buf[s, :, :])
            for s in range(K_SUB):
                s1[s].wait_send()

        @pl.when(is_inner)
        def _():
            for s in range(K_SUB):
                s1[s].wait_recv()
                p_out[s, :, :] = my_buf[s, :, :] + s1_buf[s, :, :]
                mid[s].start()
                near[s].start()
                far[s].start()
            for s in range(K_SUB):
                mid[s].wait_recv()
                ln_and_scatter(s, p_out[s, :, :] + p_in[s, :, :])
            for s in range(K_SUB):
                mid[s].wait_send()
                near[s].wait_send()
                far[s].wait_send()

        qx_r0 = (2 * (1 - my_x) + my_y) * Q
        qy_r0 = (2 * my_x + (1 - my_y)) * Q
        qd_r0 = (2 * (1 - my_x) + (1 - my_y)) * Q
        for s in range(K_SUB):
            rx[s].wait_recv()
            out_ref[pl.ds(qx_r0 + s * R, R), :] = \
                qx_buf[s, :, :].astype(jnp.float32)
            ry[s].wait_recv()
            out_ref[pl.ds(qy_r0 + s * R, R), :] = \
                qy_buf[s, :, :].astype(jnp.float32)
            rd[s].wait_recv()
            out_ref[pl.ds(qd_r0 + s * R, R), :] = \
                qd_buf[s, :, :].astype(jnp.float32)
        for s in range(K_SUB):
            rx[s].wait_send()
            ry[s].wait_send()
            rd[s].wait_send()

    sub_buf = pltpu.VMEM((K_SUB, R, d), jnp.bfloat16)
    return pl.pallas_call(
        body,
        out_shape=jax.ShapeDtypeStruct((m, d), jnp.float32),
        in_specs=[
            pl.BlockSpec(memory_space=pltpu.VMEM),
            pl.BlockSpec(memory_space=pltpu.VMEM),
            pl.BlockSpec(memory_space=pltpu.VMEM),
        ],
        out_specs=pl.BlockSpec(memory_space=pltpu.VMEM),
        scratch_shapes=[
            sub_buf,
            sub_buf,
            sub_buf,
            sub_buf,
            sub_buf,
            sub_buf,
            sub_buf,
            sub_buf,
            sub_buf,
            sub_buf,
            pltpu.SemaphoreType.DMA((K_SUB * NSEM,)),
            pltpu.SemaphoreType.DMA((K_SUB * NSEM,)),
        ],
        compiler_params=pltpu.CompilerParams(collective_id=0),
    )(partial, resid, gamma.reshape(1, d))
